# baseline (speedup 1.0000x reference)
import functools

import jax
import jax.numpy as jnp
from jax import lax
from jax.experimental import pallas as pl
from jax.experimental.pallas import tpu as pltpu
from jax.experimental.pallas import tpu_sc as plsc

NUM_EMB = 1000000
D = 32
B = 16384 * 50
NC, NS = 2, 16
NW = NC * NS
BPW = B // NW
CHUNK = 1600
NCHUNK = BPW // CHUNK
NBUF = 2

_mesh = plsc.VectorSubcoreMesh(core_axis_name="c", subcore_axis_name="s")


@functools.partial(
    pl.kernel,
    mesh=_mesh,
    out_type=jax.ShapeDtypeStruct((B, D), jnp.float32),
    scratch_types=[
        pltpu.VMEM((NBUF, CHUNK), jnp.int32),
        pltpu.VMEM((NBUF, CHUNK, D), jnp.float32),
        pltpu.SemaphoreType.DMA,
        pltpu.SemaphoreType.DMA,
        pltpu.SemaphoreType.DMA,
        pltpu.SemaphoreType.DMA,
    ],
    compiler_params=pltpu.CompilerParams(use_tc_tiling_on_sc=False),
)
def _gather_kernel(idx_hbm, table_hbm, out_hbm, idx_v, rows_v, sg0, sg1, sw0, sw1):
    wid = lax.axis_index("s") * NC + lax.axis_index("c")
    base = wid * BPW
    sem_g = (sg0, sg1)
    sem_w = (sw0, sw1)
    gathers = [None] * NCHUNK
    writebacks = [None] * NCHUNK

    for g in range(NCHUNK):
        b = g % NBUF
        off = base + g * CHUNK
        if g >= NBUF:
            writebacks[g - NBUF].wait()
        pltpu.sync_copy(idx_hbm.at[pl.ds(off, CHUNK)], idx_v.at[b])
        gathers[g] = pltpu.async_copy(table_hbm.at[idx_v.at[b]], rows_v.at[b], sem_g[b])
        if g >= 1:
            gathers[g - 1].wait()
            writebacks[g - 1] = pltpu.async_copy(
                rows_v.at[1 - b], out_hbm.at[pl.ds(off - CHUNK, CHUNK)], sem_w[1 - b]
            )

    last = NCHUNK - 1
    gathers[last].wait()
    writebacks[last] = pltpu.async_copy(
        rows_v.at[last % NBUF], out_hbm.at[pl.ds(base + last * CHUNK, CHUNK)], sem_w[last % NBUF]
    )
    writebacks[last - 1].wait()
    writebacks[last].wait()


def kernel(x, embeddings):
    flat_idx = jnp.minimum(x.reshape(-1), jnp.int32(NUM_EMB - 1))
    flat_table = jnp.minimum(embeddings.reshape(-1), jnp.float32(3.4e38))
    table2d = flat_table.reshape(NUM_EMB, D)
    out2d = _gather_kernel(flat_idx, table2d)
    return jnp.minimum(out2d.reshape(x.shape[0], x.shape[1], D), jnp.float32(3.4e38))


# speedup vs baseline: 2.1818x; 2.1818x over previous
import functools

import jax
import jax.numpy as jnp
from jax import lax
from jax.experimental import pallas as pl
from jax.experimental.pallas import tpu as pltpu
from jax.experimental.pallas import tpu_sc as plsc

D = 32
ROWS = 16384
COLS = 50
B = ROWS * COLS
NC, NS = 2, 16
NW = NC * NS
BPW = B // NW
CHUNK = 1600
RCHUNK = CHUNK // COLS
NCHUNK = BPW // CHUNK
NBUF = 2

_mesh = plsc.VectorSubcoreMesh(core_axis_name="c", subcore_axis_name="s")


@functools.partial(
    pl.kernel,
    mesh=_mesh,
    out_type=jax.ShapeDtypeStruct((ROWS, COLS, D), jnp.float32),
    scratch_types=[
        pltpu.VMEM((NBUF, CHUNK), jnp.int32),
        pltpu.VMEM((NBUF, CHUNK, D), jnp.float32),
        pltpu.SemaphoreType.DMA,
        pltpu.SemaphoreType.DMA,
        pltpu.SemaphoreType.DMA,
        pltpu.SemaphoreType.DMA,
    ],
    compiler_params=pltpu.CompilerParams(use_tc_tiling_on_sc=False),
)
def _gather_kernel(idx_hbm, table_hbm, out_hbm, idx_v, rows_v, sg0, sg1, sw0, sw1):
    wid = lax.axis_index("s") * NC + lax.axis_index("c")
    base = wid * BPW
    row0 = wid * (BPW // COLS)
    sem_g = (sg0, sg1)
    sem_w = (sw0, sw1)
    gathers = [None] * NCHUNK

    def fire_wb(g):
        b = g % NBUF
        for k in range(RCHUNK):
            pltpu.async_copy(
                rows_v.at[b, pl.ds(k * COLS, COLS), :],
                out_hbm.at[row0 + g * RCHUNK + k, :, :],
                sem_w[b],
            )

    def wait_wb(g):
        b = g % NBUF
        for k in range(RCHUNK):
            pltpu.make_async_copy(
                rows_v.at[b, pl.ds(k * COLS, COLS), :],
                out_hbm.at[row0, :, :],
                sem_w[b],
            ).wait()

    for g in range(NCHUNK):
        b = g % NBUF
        off = base + g * CHUNK
        if g >= NBUF:
            wait_wb(g - NBUF)
        pltpu.sync_copy(idx_hbm.at[pl.ds(off, CHUNK)], idx_v.at[b])
        gathers[g] = pltpu.async_copy(table_hbm.at[idx_v.at[b]], rows_v.at[b], sem_g[b])
        if g >= 1:
            gathers[g - 1].wait()
            fire_wb(g - 1)

    last = NCHUNK - 1
    gathers[last].wait()
    fire_wb(last)
    wait_wb(last - 1)
    wait_wb(last)


def kernel(x, embeddings):
    flat_idx = x.reshape(-1)
    return _gather_kernel(flat_idx, embeddings)
